# baseline (speedup 1.0000x reference)
import functools

import jax
import jax.numpy as jnp
from jax import lax
from jax.experimental import pallas as pl
from jax.experimental.pallas import tpu as pltpu
from jax.experimental.pallas import tpu_sc as plsc

LEVELS = 16
NLUT = 9
BASE_RESO = 16
TABLE_SIZE = 524288
ROWS8 = TABLE_SIZE // 4
MASK = 524287
PRIME32 = -1640531535
B = 4
N = 65536
BN = B * N
NW = 32
PER_W = BN // NW
C = 1024
NSUB = PER_W // C
JROWS = C // 128
G16 = C // 16
NHL = LEVELS - NLUT

LUT_SIZES = [BASE_RESO * (1 << l) for l in range(NLUT)]
LUT_PSIZES = [(sz + C - 1) // C * C for sz in LUT_SIZES]
LUT_OFFS = [sum(LUT_PSIZES[:l]) for l in range(NLUT)]
LUT_PAD = sum(LUT_PSIZES)


def _body(px_hbm, table_hbm, out_hbm, pxbuf, idx_v, sub_v, gb,
          obuf, lutA, lutB, bidx, bsub, gsem, psem):
    wid = lax.axis_index("s") * jnp.int32(2) + lax.axis_index("c")
    wbase = wid * jnp.int32(PER_W)
    roff8 = (wbase // jnp.int32(N)) * jnp.int32(ROWS8)

    scales = [jnp.float32(BASE_RESO * (1 << l) - 1) for l in range(LEVELS)]
    iota = lax.iota(jnp.int32, 16)

    def hash16(c0):
        return (c0 ^ (c0 * jnp.int32(PRIME32))) & jnp.int32(MASK)

    zero16 = iota * jnp.int32(0)
    for l in range(NLUT):
        size = LUT_SIZES[l]
        off = LUT_OFFS[l]
        nch = LUT_PSIZES[l] // C

        def build_chunk(ck, _, size=size, off=off):
            eb = ck * jnp.int32(C)

            def mk(g, _):
                c = jnp.minimum(eb + g * jnp.int32(16) + iota,
                                jnp.int32(size - 1))
                h = hash16(c)
                bidx[g >> jnp.int32(3),
                     pl.ds((g & jnp.int32(7)) * jnp.int32(16), 16)] = (
                    (h >> jnp.int32(2)) + roff8)
                bsub[pl.ds(g * jnp.int32(16), 16)] = (
                    (h & jnp.int32(3)) << jnp.int32(1))
                return 0

            lax.fori_loop(0, G16, mk, 0)

            def fg(j, _):
                pltpu.make_async_copy(
                    table_hbm.at[bidx.at[j]],
                    gb.at[0, pl.ds(j * jnp.int32(128), 128)],
                    gsem,
                ).start()
                return 0

            lax.fori_loop(0, JROWS, fg, 0)

            def dg(j, _):
                pltpu.make_async_copy(
                    table_hbm.at[bidx.at[j]],
                    gb.at[0, pl.ds(j * jnp.int32(128), 128)],
                    gsem,
                ).wait()
                return 0

            lax.fori_loop(0, JROWS, dg, 0)

            def ex(g, _):
                gbase = g * jnp.int32(16)
                sub = bsub[pl.ds(gbase, 16)]
                rows = gbase + iota
                v0 = plsc.load_gather(gb, [zero16, rows, sub])
                v1 = plsc.load_gather(gb, [zero16, rows, sub + jnp.int32(1)])
                dst = jnp.int32(off) + eb + gbase
                lutA[pl.ds(dst, 16)] = v0
                lutB[pl.ds(dst, 16)] = v1
                return 0

            lax.fori_loop(0, G16, ex, 0)
            return 0

        lax.fori_loop(0, jnp.int32(nch), build_chunk, 0)

    def px_copy(s):
        return pltpu.make_async_copy(
            px_hbm.at[pl.ds(wbase + s * jnp.int32(C), C)],
            pxbuf.at[s & jnp.int32(1)],
            psem,
        )

    px_copy(jnp.int32(0)).start()

    def sub_chunk(s, _):
        base = wbase + s * jnp.int32(C)
        sel = s & jnp.int32(1)
        px_copy(s).wait()

        @pl.when(s < jnp.int32(NSUB - 1))
        def _():
            px_copy(s + jnp.int32(1)).start()

        def idx_g(g, _):
            j = g >> jnp.int32(3)
            k = (g & jnp.int32(7)) * jnp.int32(16)
            x = pxbuf[sel, pl.ds(g * jnp.int32(16), 16)]
            for li in range(NHL):
                h = hash16((x * scales[NLUT + li]).astype(jnp.int32))
                idx_v[li, j, pl.ds(k, 16)] = (h >> jnp.int32(2)) + roff8
                sub_v[li, pl.ds(g * jnp.int32(16), 16)] = (
                    (h & jnp.int32(3)) << jnp.int32(1))
            return 0

        lax.fori_loop(0, G16, idx_g, 0)

        def fire(li):
            def go(j, _):
                pltpu.make_async_copy(
                    table_hbm.at[idx_v.at[li, j]],
                    gb.at[li % 2, pl.ds(j * jnp.int32(128), 128)],
                    gsem,
                ).start()
                return 0
            lax.fori_loop(0, JROWS, go, 0)

        def drain(li):
            def dr(j, _):
                pltpu.make_async_copy(
                    table_hbm.at[idx_v.at[li, j]],
                    gb.at[li % 2, pl.ds(j * jnp.int32(128), 128)],
                    gsem,
                ).wait()
                return 0
            lax.fori_loop(0, JROWS, dr, 0)

        def extract(li):
            l = NLUT + li
            lsel = iota * jnp.int32(0) + jnp.int32(li % 2)

            def ex(g, _):
                gbase = g * jnp.int32(16)
                sub = sub_v[li, pl.ds(gbase, 16)]
                rows = gbase + iota
                v0 = plsc.load_gather(gb, [lsel, rows, sub])
                v1 = plsc.load_gather(gb, [lsel, rows, sub + jnp.int32(1)])
                obuf[l, 0, pl.ds(gbase, 16)] = v0
                obuf[l, 1, pl.ds(gbase, 16)] = v1
                return 0

            lax.fori_loop(0, G16, ex, 0)

        def lut_levels():
            def ex(g, _):
                gbase = g * jnp.int32(16)
                x = pxbuf[sel, pl.ds(gbase, 16)]
                for l in range(NLUT):
                    c0 = (x * scales[l]).astype(jnp.int32)
                    e = c0 + jnp.int32(LUT_OFFS[l])
                    v0 = plsc.load_gather(lutA, [e])
                    v1 = plsc.load_gather(lutB, [e])
                    obuf[l, 0, pl.ds(gbase, 16)] = v0
                    obuf[l, 1, pl.ds(gbase, 16)] = v1
                return 0

            lax.fori_loop(0, G16, ex, 0)

        fire(0)
        fire(1)
        lut_levels()
        drain(0)
        extract(0)
        for li in range(2, NHL):
            fire(li)
            drain(li - 1)
            extract(li - 1)
        drain(NHL - 1)
        extract(NHL - 1)

        pltpu.sync_copy(obuf, out_hbm.at[:, :, pl.ds(base, C)])
        return 0

    lax.fori_loop(0, NSUB, sub_chunk, 0)


@jax.jit
def _run(p, enc):
    px = p[..., 0].reshape(BN)
    table = enc.reshape(B * ROWS8, 8)
    mesh = plsc.VectorSubcoreMesh(core_axis_name="c", subcore_axis_name="s")
    run = functools.partial(
        pl.kernel,
        mesh=mesh,
        out_type=jax.ShapeDtypeStruct((LEVELS, 2, BN), jnp.float32),
        scratch_types=[
            pltpu.VMEM((2, C), jnp.float32),
            pltpu.VMEM((NHL, JROWS, 128), jnp.int32),
            pltpu.VMEM((NHL, C), jnp.int32),
            pltpu.VMEM((2, C, 8), jnp.float32),
            pltpu.VMEM((LEVELS, 2, C), jnp.float32),
            pltpu.VMEM((LUT_PAD,), jnp.float32),
            pltpu.VMEM((LUT_PAD,), jnp.float32),
            pltpu.VMEM((JROWS, 128), jnp.int32),
            pltpu.VMEM((C,), jnp.int32),
            pltpu.SemaphoreType.DMA,
            pltpu.SemaphoreType.DMA,
        ],
        compiler_params=pltpu.CompilerParams(
            use_tc_tiling_on_sc=False, needs_layout_passes=False),
    )(_body)
    out = run(px, table)
    return out.transpose(2, 0, 1).reshape(B, N, LEVELS * 2)


def kernel(p, enc):
    with jax.enable_x64(False):
        return _run(p, enc)
